# Initial kernel scaffold; baseline (speedup 1.0000x reference)
import jax
import jax.numpy as jnp
from jax.experimental import pallas as pl


def kernel(x, emb):
    raise NotImplementedError("write your pallas kernel here")


# speedup vs baseline: 1.7347x; 1.7347x over previous
import functools

import jax
import jax.numpy as jnp
from jax import lax
from jax.experimental import pallas as pl
from jax.experimental.pallas import tpu as pltpu
from jax.experimental.pallas import tpu_sc as plsc

VOCAB = 8192
B, T = 8, 512
N = B * T
NW = 32
ROWS_PER_W = N // NW
CHUNK = 8
NCHUNK = ROWS_PER_W // CHUNK

_mesh = plsc.VectorSubcoreMesh(core_axis_name="c", subcore_axis_name="s")


@functools.partial(
    pl.kernel,
    out_type=jax.ShapeDtypeStruct((N, VOCAB), jnp.float32),
    mesh=_mesh,
    scratch_types=[
        pltpu.VMEM((NCHUNK, CHUNK), jnp.int32),
        pltpu.VMEM((CHUNK, VOCAB), jnp.float32),
        pltpu.SemaphoreType.DMA,
    ],
)
def _gather_sc(idx_hbm, emb_hbm, out_hbm, idx_v, rows_v, sem):
    wid = lax.axis_index("s") * 2 + lax.axis_index("c")
    base = wid * ROWS_PER_W
    pltpu.sync_copy(idx_hbm.at[pl.ds(wid * NCHUNK, NCHUNK)], idx_v)

    def body(c, carry):
        pltpu.async_copy(emb_hbm.at[idx_v.at[c]], rows_v, sem).wait()
        pltpu.sync_copy(rows_v, out_hbm.at[pl.ds(base + c * CHUNK, CHUNK)])
        return carry

    lax.fori_loop(0, NCHUNK, body, 0)


def kernel(x, emb):
    idx2d = x.reshape(N // CHUNK, CHUNK)
    out = _gather_sc(idx2d, emb)
    return out.reshape(B, T, VOCAB)


# speedup vs baseline: 1.8562x; 1.0700x over previous
import functools

import jax
import jax.numpy as jnp
from jax import lax
from jax.experimental import pallas as pl
from jax.experimental.pallas import tpu as pltpu
from jax.experimental.pallas import tpu_sc as plsc

VOCAB = 8192
B, T = 8, 512
N = B * T
NW = 32
ROWS_PER_W = N // NW
NBUF = 8
NBLK = ROWS_PER_W // NBUF

_mesh = plsc.VectorSubcoreMesh(core_axis_name="c", subcore_axis_name="s")


@functools.partial(
    pl.kernel,
    out_type=jax.ShapeDtypeStruct((N, VOCAB), jnp.float32),
    mesh=_mesh,
    scratch_types=[
        pltpu.VMEM((ROWS_PER_W, 1), jnp.int32),
        pltpu.VMEM((NBUF, 1, VOCAB), jnp.float32),
        pltpu.SemaphoreType.DMA((NBUF,)),
        pltpu.SemaphoreType.DMA((NBUF,)),
    ],
)
def _gather_sc(idx_hbm, emb_hbm, out_hbm, idx_v, rows_v, gsem, ssem):
    wid = lax.axis_index("s") * 2 + lax.axis_index("c")
    base = wid * ROWS_PER_W
    pltpu.sync_copy(idx_hbm.at[pl.ds(wid * ROWS_PER_W, ROWS_PER_W)], idx_v)

    def gather(c, b):
        pltpu.async_copy(emb_hbm.at[idx_v.at[c]], rows_v.at[b], gsem.at[b])

    def store(c, b):
        pltpu.async_copy(rows_v.at[b], out_hbm.at[pl.ds(base + c, 1)],
                         ssem.at[b])

    def wait_g(b):
        pltpu.make_async_copy(emb_hbm.at[pl.ds(0, 1)], rows_v.at[b],
                              gsem.at[b]).wait()

    def wait_s(b):
        pltpu.make_async_copy(rows_v.at[b], out_hbm.at[pl.ds(base, 1)],
                              ssem.at[b]).wait()

    for b in range(NBUF):
        gather(b, b)

    def body(k, carry):
        c0 = k * NBUF
        for b in range(NBUF):
            wait_g(b)
            store(c0 + b, b)
        for b in range(NBUF):
            wait_s(b)
            gather(c0 + NBUF + b, b)
        return carry

    lax.fori_loop(0, NBLK - 1, body, 0)

    c0 = (NBLK - 1) * NBUF
    for b in range(NBUF):
        wait_g(b)
        store(c0 + b, b)
    for b in range(NBUF):
        wait_s(b)


def kernel(x, emb):
    idx2d = x.reshape(N, 1)
    out = _gather_sc(idx2d, emb)
    return out.reshape(B, T, VOCAB)


# speedup vs baseline: 1.8633x; 1.0038x over previous
import functools

import jax
import jax.numpy as jnp
from jax import lax
from jax.experimental import pallas as pl
from jax.experimental.pallas import tpu as pltpu
from jax.experimental.pallas import tpu_sc as plsc

VOCAB = 8192
B, T = 8, 512
N = B * T
NW = 32
ROWS_PER_W = N // NW
K = 2
NBUF = 4
NCHUNK = ROWS_PER_W // K
NBLK = NCHUNK // NBUF

_mesh = plsc.VectorSubcoreMesh(core_axis_name="c", subcore_axis_name="s")


@functools.partial(
    pl.kernel,
    out_type=jax.ShapeDtypeStruct((N, VOCAB), jnp.float32),
    mesh=_mesh,
    scratch_types=[
        pltpu.VMEM((NCHUNK, K), jnp.int32),
        pltpu.VMEM((NBUF, K, VOCAB), jnp.float32),
        pltpu.SemaphoreType.DMA((NBUF,)),
        pltpu.SemaphoreType.DMA((NBUF,)),
    ],
)
def _gather_sc(idx_hbm, emb_hbm, out_hbm, idx_v, rows_v, gsem, ssem):
    wid = lax.axis_index("s") * 2 + lax.axis_index("c")
    base = wid * ROWS_PER_W
    pltpu.sync_copy(idx_hbm.at[pl.ds(wid * NCHUNK, NCHUNK)], idx_v)

    def gather(c, b):
        pltpu.async_copy(emb_hbm.at[idx_v.at[c]], rows_v.at[b], gsem.at[b])

    def store(c, b):
        pltpu.async_copy(rows_v.at[b], out_hbm.at[pl.ds(base + c * K, K)],
                         ssem.at[b])

    def wait_g(b):
        pltpu.make_async_copy(emb_hbm.at[pl.ds(0, K)], rows_v.at[b],
                              gsem.at[b]).wait()

    def wait_s(b):
        pltpu.make_async_copy(rows_v.at[b], out_hbm.at[pl.ds(base, K)],
                              ssem.at[b]).wait()

    for b in range(NBUF):
        gather(b, b)

    def body(k, carry):
        c0 = k * NBUF
        for b in range(NBUF):
            wait_g(b)
            store(c0 + b, b)
        for b in range(NBUF):
            wait_s(b)
            gather(c0 + NBUF + b, b)
        return carry

    lax.fori_loop(0, NBLK - 1, body, 0)

    c0 = (NBLK - 1) * NBUF
    for b in range(NBUF):
        wait_g(b)
        store(c0 + b, b)
    for b in range(NBUF):
        wait_s(b)


def kernel(x, emb):
    idx2d = x.reshape(NCHUNK * NW, K)
    out = _gather_sc(idx2d, emb)
    return out.reshape(B, T, VOCAB)
